# baseline (speedup 1.0000x reference)
import functools

import jax
import jax.numpy as jnp
from jax import lax
from jax.experimental import pallas as pl
from jax.experimental.pallas import tpu as pltpu
from jax.experimental.pallas import tpu_sc as plsc

VOCAB = 100000
EMBED = 1000
BATCH = 1024
VB = 2048

_NROWS = EMBED // 8
_WIN = 6272
_MAIN = (VOCAB // 128) * 128
_TAIL = VOCAB - _MAIN


def _win_sizes():
    sizes, off = [], 0
    while off < _MAIN:
        w = min(_WIN, _MAIN - off)
        sizes.append(w)
        off += w
    return sizes


def _gather_sc(table_t, tail_t, idx):
    info = plsc.get_sparse_core_info()
    nw = info.num_cores * info.num_subcores
    mesh = plsc.VectorSubcoreMesh(core_axis_name="c", subcore_axis_name="s")

    @functools.partial(
        pl.kernel,
        mesh=mesh,
        out_type=jax.ShapeDtypeStruct((EMBED, BATCH), jnp.float32),
        scratch_types=[
            pltpu.VMEM((1024,), jnp.int32),
            pltpu.VMEM((8, _WIN), jnp.float32),
            pltpu.VMEM((8, _WIN), jnp.float32),
            pltpu.VMEM((8, _TAIL), jnp.float32),
            pltpu.VMEM((8, BATCH), jnp.float32),
            pltpu.SemaphoreType.DMA,
            pltpu.SemaphoreType.DMA,
        ],
        compiler_params=pltpu.CompilerParams(needs_layout_passes=False),
    )
    def k(
        table_hbm, tail_hbm, idx_hbm, out_hbm,
        idx_v, buf_a, buf_b, tbuf_v, out_v, sem_a, sem_b,
    ):
        wid = lax.axis_index("s") * info.num_cores + lax.axis_index("c")
        pltpu.sync_copy(idx_hbm, idx_v)
        r_lo = wid * _NROWS // nw
        r_hi = (wid + 1) * _NROWS // nw
        bufs = [(buf_a, sem_a), (buf_b, sem_b)]
        wins = _win_sizes()
        offs = [sum(wins[:i]) for i in range(len(wins))]

        def win_copy(r, w):
            buf, sem = bufs[w % 2]
            return pltpu.make_async_copy(
                table_hbm.at[pl.ds(r * 8, 8), pl.ds(offs[w], wins[w])],
                buf.at[:, pl.ds(0, wins[w])],
                sem,
            )

        def extract(buf, lo, wsz):
            def do_vec(v, c):
                xv = idx_v[pl.ds(v * 16, 16)]
                rel = xv - lo
                m = (rel >= 0) & (rel < wsz)
                relc = jnp.minimum(jnp.maximum(rel, 0), wsz - 1)
                jpos = lax.iota(jnp.int32, 16) + v * 16
                for s in range(8):
                    sv = jnp.full((16,), s, jnp.int32)
                    vals = plsc.load_gather(buf, [sv, relc], mask=m)
                    vals = jnp.maximum(vals, 0.0)
                    plsc.store_scatter(out_v, [sv, jpos], vals, mask=m)
                return c

            lax.fori_loop(0, BATCH // 16, do_vec, 0)

        def do_row(r, carry):
            win_copy(r, 0).start()
            for w in range(len(wins)):
                if w + 1 < len(wins):
                    win_copy(r, w + 1).start()
                win_copy(r, w).wait()
                extract(bufs[w % 2][0], offs[w], wins[w])
            pltpu.sync_copy(tail_hbm.at[pl.ds(r * 8, 8), :], tbuf_v)
            extract(tbuf_v, _MAIN, _TAIL)
            pltpu.sync_copy(out_v, out_hbm.at[pl.ds(r * 8, 8), :])
            return carry

        lax.fori_loop(r_lo, r_hi, do_row, 0)

    return k(table_t, tail_t, idx)


def _mm_body(wt_ref, h_ref, b_ref, o_ref):
    o_ref[...] = (
        lax.dot_general(
            wt_ref[...],
            h_ref[...],
            (((0,), (0,)), ((), ())),
            preferred_element_type=jnp.float32,
        )
        + jnp.transpose(b_ref[...], (1, 0))
    )


def _matmul_tc(w_t, e_t, b2):
    nb = (VOCAB + VB - 1) // VB
    return pl.pallas_call(
        _mm_body,
        grid=(nb,),
        in_specs=[
            pl.BlockSpec((EMBED, VB), lambda i: (0, i)),
            pl.BlockSpec((EMBED, BATCH), lambda i: (0, 0)),
            pl.BlockSpec((1, VB), lambda i: (0, i)),
        ],
        out_specs=pl.BlockSpec((VB, BATCH), lambda i: (i, 0)),
        out_shape=jax.ShapeDtypeStruct((VOCAB, BATCH), jnp.float32),
    )(w_t, e_t, b2)


def kernel(x, table, W, b):
    x = x.astype(jnp.int32)
    table_t = table.T
    e_t = _gather_sc(table_t, table_t[:, _MAIN:], x)
    logits_t = _matmul_tc(W.T, e_t, b.reshape(1, VOCAB))
    return logits_t.T


# speedup vs baseline: 1.0527x; 1.0527x over previous
import functools

import jax
import jax.numpy as jnp
from jax import lax
from jax.experimental import pallas as pl
from jax.experimental.pallas import tpu as pltpu
from jax.experimental.pallas import tpu_sc as plsc

VOCAB = 100000
EMBED = 1000
BATCH = 1024
VB = 2048

_NROWS = EMBED // 8
_WIN = 6272
_MAIN = (VOCAB // 128) * 128
_TAIL = VOCAB - _MAIN


def _win_sizes():
    sizes, off = [], 0
    while off < _MAIN:
        w = min(_WIN, _MAIN - off)
        sizes.append(w)
        off += w
    return sizes


def _gather_sc(table_t, tail_t, idx):
    info = plsc.get_sparse_core_info()
    nw = info.num_cores * info.num_subcores
    mesh = plsc.VectorSubcoreMesh(core_axis_name="c", subcore_axis_name="s")

    @functools.partial(
        pl.kernel,
        mesh=mesh,
        out_type=jax.ShapeDtypeStruct((EMBED, BATCH), jnp.float32),
        scratch_types=[
            pltpu.VMEM((1024,), jnp.int32),
            pltpu.VMEM((8, _WIN), jnp.float32),
            pltpu.VMEM((8, _WIN), jnp.float32),
            pltpu.VMEM((8, _TAIL), jnp.float32),
            pltpu.VMEM((8, BATCH), jnp.float32),
            pltpu.SemaphoreType.DMA,
            pltpu.SemaphoreType.DMA,
        ],
        compiler_params=pltpu.CompilerParams(needs_layout_passes=False),
    )
    def k(
        table_hbm, tail_hbm, idx_hbm, out_hbm,
        idx_v, buf_a, buf_b, tbuf_v, out_v, sem_a, sem_b,
    ):
        wid = lax.axis_index("s") * info.num_cores + lax.axis_index("c")
        pltpu.sync_copy(idx_hbm, idx_v)
        r_lo = wid * _NROWS // nw
        r_hi = (wid + 1) * _NROWS // nw
        bufs = [(buf_a, sem_a), (buf_b, sem_b)]
        wins = _win_sizes()
        offs = [sum(wins[:i]) for i in range(len(wins))]

        def win_copy(r, w):
            buf, sem = bufs[w % 2]
            return pltpu.make_async_copy(
                table_hbm.at[pl.ds(r * 8, 8), pl.ds(offs[w], wins[w])],
                buf.at[:, pl.ds(0, wins[w])],
                sem,
            )

        def extract(buf, lo, wsz):
            def do_vec(v, c):
                xv = idx_v[pl.ds(v * 16, 16)]
                rel = xv - lo
                m = (rel >= 0) & (rel < wsz)
                relc = jnp.minimum(jnp.maximum(rel, 0), wsz - 1)
                jpos = lax.iota(jnp.int32, 16) + v * 16
                for s in range(8):
                    sv = jnp.full((16,), s, jnp.int32)
                    vals = plsc.load_gather(buf, [sv, relc], mask=m)
                    plsc.store_scatter(out_v, [sv, jpos], vals, mask=m)
                return c

            lax.fori_loop(0, BATCH // 16, do_vec, 0)

        def do_row(r, carry):
            win_copy(r, 0).start()
            for w in range(len(wins)):
                if w + 1 < len(wins):
                    win_copy(r, w + 1).start()
                win_copy(r, w).wait()
                extract(bufs[w % 2][0], offs[w], wins[w])
            pltpu.sync_copy(tail_hbm.at[pl.ds(r * 8, 8), :], tbuf_v)
            extract(tbuf_v, _MAIN, _TAIL)
            pltpu.sync_copy(out_v, out_hbm.at[pl.ds(r * 8, 8), :])
            return carry

        lax.fori_loop(r_lo, r_hi, do_row, 0)

    return k(table_t, tail_t, idx)


def _mm_body(wt_ref, et_ref, b_ref, o_ref, h_ref):
    @pl.when(pl.program_id(0) == 0)
    def _():
        h_ref[...] = jnp.maximum(et_ref[...], 0.0)

    o_ref[...] = (
        lax.dot_general(
            wt_ref[...],
            h_ref[...],
            (((0,), (0,)), ((), ())),
            preferred_element_type=jnp.float32,
        )
        + jnp.transpose(b_ref[...], (1, 0))
    )


def _matmul_tc(w_t, e_t, b2):
    nb = (VOCAB + VB - 1) // VB
    return pl.pallas_call(
        _mm_body,
        grid=(nb,),
        in_specs=[
            pl.BlockSpec((EMBED, VB), lambda i: (0, i)),
            pl.BlockSpec((EMBED, BATCH), lambda i: (0, 0)),
            pl.BlockSpec((1, VB), lambda i: (0, i)),
        ],
        out_specs=pl.BlockSpec((VB, BATCH), lambda i: (i, 0)),
        out_shape=jax.ShapeDtypeStruct((VOCAB, BATCH), jnp.float32),
        scratch_shapes=[pltpu.VMEM((EMBED, BATCH), jnp.float32)],
    )(w_t, e_t, b2)


def kernel(x, table, W, b):
    x = x.astype(jnp.int32)
    table_t = table.T
    e_t = _gather_sc(table_t, table_t[:, _MAIN:], x)
    logits_t = _matmul_tc(W.T, e_t, b.reshape(1, VOCAB))
    return logits_t.T
